# baseline (speedup 1.0000x reference)
import jax
import jax.numpy as jnp
from jax.experimental import pallas as pl
from jax.experimental.pallas import tpu as pltpu

B, N, C, W, H = 8, 16, 64, 32, 32
TS = 64
ROWS = 4
NBUF = N // ROWS
CH = B * NBUF


def _assemble_kernel(x_ref, t_ref, emb_ref, out_ref, stage_ref, insem, outsem):
    for k in range(NBUF):
        for r in range(ROWS):
            row = emb_ref[t_ref[ROWS * k + r], :]
            stage_ref[k, r, C:] = jax.lax.broadcast_in_dim(row, (TS, W, H), (0,))

    def in_copy(c):
        b, k = divmod(c, NBUF)
        return pltpu.make_async_copy(
            x_ref.at[b, pl.ds(ROWS * k, ROWS)], stage_ref.at[k, :, 0:C],
            insem.at[k])

    def out_copy(c):
        b, k = divmod(c, NBUF)
        return pltpu.make_async_copy(
            stage_ref.at[k], out_ref.at[b, pl.ds(ROWS * k, ROWS)], outsem.at[k])

    ins = {}
    outs = {}
    for c in range(NBUF):
        ins[c] = in_copy(c)
        ins[c].start()
    for c in range(CH):
        if c >= NBUF:
            outs[c - NBUF].wait()
            ins[c] = in_copy(c)
            ins[c].start()
        ins[c].wait()
        outs[c] = out_copy(c)
        outs[c].start()
    for c in range(CH - NBUF, CH):
        outs[c].wait()


def kernel(x, t, emb_table):
    return pl.pallas_call(
        _assemble_kernel,
        in_specs=[
            pl.BlockSpec(memory_space=pl.ANY),
            pl.BlockSpec(memory_space=pltpu.SMEM),
            pl.BlockSpec(memory_space=pltpu.VMEM),
        ],
        out_specs=pl.BlockSpec(memory_space=pl.ANY),
        out_shape=jax.ShapeDtypeStruct((B, N, C + TS, W, H), x.dtype),
        scratch_shapes=[
            pltpu.VMEM((NBUF, ROWS, C + TS, W, H), x.dtype),
            pltpu.SemaphoreType.DMA((NBUF,)),
            pltpu.SemaphoreType.DMA((NBUF,)),
        ],
    )(x, t, emb_table)


# speedup vs baseline: 1.0209x; 1.0209x over previous
import jax
import jax.numpy as jnp
from jax.experimental import pallas as pl
from jax.experimental.pallas import tpu as pltpu

B, N, C, W, H = 8, 16, 64, 32, 32
TS = 64
ROWS = 8
NXBUF = 4
CH = B * (N // ROWS)


def _assemble_kernel(x_ref, t_ref, emb_ref, out_ref, xbuf_ref, tv_ref,
                     insem, outxsem, outtvsem):
    for n in range(N):
        row = emb_ref[t_ref[n], :]
        tv_ref[n] = jax.lax.broadcast_in_dim(row, (TS, W, H), (0,))

    def in_copy(c):
        b, g = divmod(c, N // ROWS)
        j = c % NXBUF
        cp = pltpu.make_async_copy(
            x_ref.at[b, pl.ds(g * ROWS, ROWS)], xbuf_ref.at[j], insem.at[j])
        cp.start(priority=c % 2)
        return cp

    def outx_copy(c):
        b, g = divmod(c, N // ROWS)
        j = c % NXBUF
        cp = pltpu.make_async_copy(
            xbuf_ref.at[j], out_ref.at[b, pl.ds(g * ROWS, ROWS), 0:C],
            outxsem.at[j])
        cp.start(priority=c % 2)
        return cp

    def outtv_copy(c):
        b, g = divmod(c, N // ROWS)
        j = c % NXBUF
        cp = pltpu.make_async_copy(
            tv_ref.at[pl.ds(g * ROWS, ROWS)],
            out_ref.at[b, pl.ds(g * ROWS, ROWS), C:], outtvsem.at[j])
        cp.start(priority=c % 2)
        return cp

    ins = {}
    outx = {}
    outtv = {}
    for c in range(NXBUF):
        ins[c] = in_copy(c)
    for c in range(CH):
        outtv[c] = outtv_copy(c)
    for c in range(CH):
        if c >= NXBUF:
            outx[c - NXBUF].wait()
            ins[c] = in_copy(c)
        ins[c].wait()
        outx[c] = outx_copy(c)
    for c in range(CH - NXBUF, CH):
        outx[c].wait()
    for c in range(CH):
        outtv[c].wait()


def kernel(x, t, emb_table):
    return pl.pallas_call(
        _assemble_kernel,
        in_specs=[
            pl.BlockSpec(memory_space=pl.ANY),
            pl.BlockSpec(memory_space=pltpu.SMEM),
            pl.BlockSpec(memory_space=pltpu.VMEM),
        ],
        out_specs=pl.BlockSpec(memory_space=pl.ANY),
        out_shape=jax.ShapeDtypeStruct((B, N, C + TS, W, H), x.dtype),
        scratch_shapes=[
            pltpu.VMEM((NXBUF, ROWS, C, W, H), x.dtype),
            pltpu.VMEM((N, TS, W, H), x.dtype),
            pltpu.SemaphoreType.DMA((NXBUF,)),
            pltpu.SemaphoreType.DMA((NXBUF,)),
            pltpu.SemaphoreType.DMA((NXBUF,)),
        ],
    )(x, t, emb_table)
